# baseline (speedup 1.0000x reference)
import functools

import jax
import jax.numpy as jnp
from jax import lax
from jax.experimental import pallas as pl
from jax.experimental.pallas import tpu as pltpu
from jax.experimental.pallas import tpu_sc as plsc

NW = 32
S = 16


def _make_sc_kernel(B, T, D):
    rows_per_w = T // NW
    n_chunks = rows_per_w // S
    buf = S * D

    mesh = plsc.VectorSubcoreMesh(core_axis_name="c", subcore_axis_name="s")

    @functools.partial(
        pl.kernel,
        mesh=mesh,
        out_type=jax.ShapeDtypeStruct((B * T * D,), jnp.float32),
        scratch_types=[
            pltpu.VMEM((buf,), jnp.float32),
            pltpu.VMEM((buf,), jnp.float32),
        ],
    )
    def k(x_hbm, emb_hbm, out_hbm, ebuf, xbuf):
        wid = lax.axis_index("s") * 2 + lax.axis_index("c")
        base_row = wid * rows_per_w

        def chunk_body(ci, carry):
            r0 = base_row + ci * S
            pltpu.sync_copy(emb_hbm.at[pl.ds(r0 * D, buf)], ebuf)
            for b in range(B):
                off = (b * T + r0) * D
                pltpu.sync_copy(x_hbm.at[pl.ds(off, buf)], xbuf)

                def add_body(i, c):
                    sl = pl.ds(i * 16, 16)
                    xbuf[sl] = xbuf[sl] + ebuf[sl]
                    return c

                lax.fori_loop(0, buf // 16, add_body, 0)
                pltpu.sync_copy(xbuf, out_hbm.at[pl.ds(off, buf)])
            return carry

        lax.fori_loop(0, n_chunks, chunk_body, 0)

    return k


def kernel(x, emb):
    B, T, D = x.shape
    k = _make_sc_kernel(B, T, D)
    out = k(x.reshape(-1), emb[:T].reshape(-1))
    return out.reshape(B, T, D)


# speedup vs baseline: 1.8713x; 1.8713x over previous
import functools

import jax
import jax.numpy as jnp
from jax import lax
from jax.experimental import pallas as pl
from jax.experimental.pallas import tpu as pltpu
from jax.experimental.pallas import tpu_sc as plsc

NW = 32
S = 16
UNROLL = 8
NX = 4


def _make_sc_kernel(B, T, D):
    rows_per_w = T // NW
    n_chunks = rows_per_w // S
    n_jobs = n_chunks * B
    buf = S * D

    mesh = plsc.VectorSubcoreMesh(core_axis_name="c", subcore_axis_name="s")

    @functools.partial(
        pl.kernel,
        mesh=mesh,
        out_type=jax.ShapeDtypeStruct((B * T * D,), jnp.float32),
        scratch_types=(
            [pltpu.VMEM((buf,), jnp.float32) for _ in range(2)]
            + [pltpu.VMEM((buf,), jnp.float32) for _ in range(NX)]
            + [pltpu.SemaphoreType.DMA for _ in range(2 + 2 * NX)]
        ),
    )
    def k(x_hbm, emb_hbm, out_hbm, *scratch):
        ebufs = scratch[0:2]
        xbufs = scratch[2:2 + NX]
        esems = scratch[2 + NX:4 + NX]
        xsems = scratch[4 + NX:4 + 2 * NX]
        osems = scratch[4 + 2 * NX:4 + 3 * NX]
        wid = lax.axis_index("s") * 2 + lax.axis_index("c")
        base_row = wid * rows_per_w

        def x_off(j):
            ci, b = divmod(j, B)
            return (b * T + base_row + ci * S) * D

        def start_x(j):
            pltpu.async_copy(
                x_hbm.at[pl.ds(x_off(j), buf)], xbufs[j % NX], xsems[j % NX])

        def start_emb(ci):
            r0 = (base_row + ci * S) * D
            pltpu.async_copy(
                emb_hbm.at[pl.ds(r0, buf)], ebufs[ci % 2], esems[ci % 2])

        def wait_out(j):
            pltpu.make_async_copy(
                xbufs[j % NX], out_hbm.at[pl.ds(x_off(j), buf)], osems[j % NX]
            ).wait()

        start_emb(0)
        start_x(0)
        start_x(1)

        for j in range(n_jobs):
            ci, b = divmod(j, B)
            xbuf, ebuf = xbufs[j % NX], ebufs[ci % 2]
            if j >= 2:
                wait_out(j - 2)
            if j + 2 < n_jobs:
                start_x(j + 2)
            nci = (j + 1) // B
            if nci != ci and nci < n_chunks:
                start_emb(nci)
            pltpu.make_async_copy(
                x_hbm.at[pl.ds(x_off(j), buf)], xbuf, xsems[j % NX]).wait()
            if b == 0:
                pltpu.make_async_copy(
                    emb_hbm.at[pl.ds((base_row + ci * S) * D, buf)],
                    ebuf, esems[ci % 2]).wait()

            def add_body(i, c, xbuf=xbuf, ebuf=ebuf):
                for u in range(UNROLL):
                    sl = pl.ds((i * UNROLL + u) * 16, 16)
                    plsc.addupdate(xbuf.at[sl], ebuf[sl])
                return c

            lax.fori_loop(0, buf // (16 * UNROLL), add_body, 0)

            pltpu.async_copy(
                xbuf, out_hbm.at[pl.ds(x_off(j), buf)], osems[j % NX])

        wait_out(n_jobs - 2)
        wait_out(n_jobs - 1)

    return k


def kernel(x, emb):
    B, T, D = x.shape
    k = _make_sc_kernel(B, T, D)
    out = k(x.reshape(-1), emb[:T].reshape(-1))
    return out.reshape(B, T, D)


# speedup vs baseline: 2.3893x; 1.2768x over previous
import functools

import jax
import jax.numpy as jnp
from jax import lax
from jax.experimental import pallas as pl
from jax.experimental.pallas import tpu as pltpu
from jax.experimental.pallas import tpu_sc as plsc

NW = 32
S = 16
UNROLL = 8
NX = 4


def _make_sc_kernel(B, T, D):
    rows_per_w = T // NW
    n_chunks = rows_per_w // S
    n_jobs = n_chunks * B

    mesh = plsc.VectorSubcoreMesh(core_axis_name="c", subcore_axis_name="s")

    @functools.partial(
        pl.kernel,
        mesh=mesh,
        out_type=jax.ShapeDtypeStruct((B, T, D), jnp.float32),
        scratch_types=(
            [pltpu.VMEM((S, D), jnp.float32) for _ in range(2)]
            + [pltpu.VMEM((S, D), jnp.float32) for _ in range(NX)]
            + [pltpu.SemaphoreType.DMA for _ in range(2 + 2 * NX)]
        ),
    )
    def k(x_hbm, emb_hbm, out_hbm, *scratch):
        ebufs = scratch[0:2]
        xbufs = scratch[2:2 + NX]
        esems = scratch[2 + NX:4 + NX]
        xsems = scratch[4 + NX:4 + 2 * NX]
        osems = scratch[4 + 2 * NX:4 + 3 * NX]
        wid = lax.axis_index("s") * 2 + lax.axis_index("c")
        base_row = wid * rows_per_w

        def rows(j):
            ci, b = divmod(j, B)
            return b, base_row + ci * S

        def start_x(j):
            b, r0 = rows(j)
            pltpu.async_copy(
                x_hbm.at[b, pl.ds(r0, S), :], xbufs[j % NX], xsems[j % NX])

        def start_emb(ci):
            pltpu.async_copy(
                emb_hbm.at[pl.ds(base_row + ci * S, S), :],
                ebufs[ci % 2], esems[ci % 2])

        def wait_out(j):
            b, r0 = rows(j)
            pltpu.make_async_copy(
                xbufs[j % NX], out_hbm.at[b, pl.ds(r0, S), :], osems[j % NX]
            ).wait()

        start_emb(0)
        start_x(0)
        start_x(1)

        for j in range(n_jobs):
            ci, b = divmod(j, B)
            xbuf, ebuf = xbufs[j % NX], ebufs[ci % 2]
            if j >= 2:
                wait_out(j - 2)
            if j + 2 < n_jobs:
                start_x(j + 2)
            nci = (j + 1) // B
            if nci != ci and nci < n_chunks:
                start_emb(nci)
            b_, r0 = rows(j)
            pltpu.make_async_copy(
                x_hbm.at[b_, pl.ds(r0, S), :], xbuf, xsems[j % NX]).wait()
            if b == 0:
                pltpu.make_async_copy(
                    emb_hbm.at[pl.ds(r0, S), :], ebuf, esems[ci % 2]).wait()

            def radd(r, c, xbuf=xbuf, ebuf=ebuf):
                def cadd(i, c2):
                    for u in range(UNROLL):
                        sl = pl.ds((i * UNROLL + u) * 16, 16)
                        plsc.addupdate(xbuf.at[r, sl], ebuf[r, sl])
                    return c2

                return lax.fori_loop(0, D // (16 * UNROLL), cadd, c)

            lax.fori_loop(0, S, radd, 0)

            pltpu.async_copy(
                xbuf, out_hbm.at[b_, pl.ds(r0, S), :], osems[j % NX])

        wait_out(n_jobs - 2)
        wait_out(n_jobs - 1)

    return k


def kernel(x, emb):
    B, T, D = x.shape
    k = _make_sc_kernel(B, T, D)
    return k(x, emb[:T])


# speedup vs baseline: 5.4796x; 2.2934x over previous
import functools

import jax
import jax.numpy as jnp
from jax import lax
from jax.experimental import pallas as pl
from jax.experimental.pallas import tpu as pltpu
from jax.experimental.pallas import tpu_sc as plsc

NW = 32
S = 16
UNROLL = 8
NX = 4
T1 = 6144
BT = 2048


def _add_block(x_ref, emb_ref, o_ref):
    o_ref[...] = x_ref[...] + emb_ref[...]


def _tc_part(x, emb, B, T, D):
    grid = (T1 // BT, B)
    return pl.pallas_call(
        _add_block,
        grid=grid,
        in_specs=[
            pl.BlockSpec((1, BT, D), lambda t, b: (b, t, 0)),
            pl.BlockSpec((BT, D), lambda t, b: (t, 0)),
        ],
        out_specs=pl.BlockSpec((1, BT, D), lambda t, b: (b, t, 0)),
        out_shape=jax.ShapeDtypeStruct((B, T, D), x.dtype),
        compiler_params=pltpu.CompilerParams(
            dimension_semantics=("parallel", "parallel"),
        ),
    )(x, emb)


def _make_sc_part(B, T, D):
    t_rows = T - T1
    rows_per_w = t_rows // NW
    n_chunks = rows_per_w // S
    n_jobs = n_chunks * B

    mesh = plsc.VectorSubcoreMesh(core_axis_name="c", subcore_axis_name="s")

    @functools.partial(
        pl.kernel,
        mesh=mesh,
        out_type=jax.ShapeDtypeStruct((B, t_rows, D), jnp.float32),
        scratch_types=(
            [pltpu.VMEM((S, D), jnp.float32) for _ in range(2)]
            + [pltpu.VMEM((S, D), jnp.float32) for _ in range(NX)]
            + [pltpu.SemaphoreType.DMA for _ in range(2 + 2 * NX)]
        ),
    )
    def k(x_hbm, emb_hbm, out_hbm, *scratch):
        ebufs = scratch[0:2]
        xbufs = scratch[2:2 + NX]
        esems = scratch[2 + NX:4 + NX]
        xsems = scratch[4 + NX:4 + 2 * NX]
        osems = scratch[4 + 2 * NX:4 + 3 * NX]
        wid = lax.axis_index("s") * 2 + lax.axis_index("c")
        base_row = wid * rows_per_w

        def rows(j):
            ci, b = divmod(j, B)
            return b, base_row + ci * S

        def start_x(j):
            b, r0 = rows(j)
            pltpu.async_copy(
                x_hbm.at[b, pl.ds(T1 + r0, S), :], xbufs[j % NX],
                xsems[j % NX])

        def start_emb(ci):
            pltpu.async_copy(
                emb_hbm.at[pl.ds(T1 + base_row + ci * S, S), :],
                ebufs[ci % 2], esems[ci % 2])

        def wait_out(j):
            b, r0 = rows(j)
            pltpu.make_async_copy(
                xbufs[j % NX], out_hbm.at[b, pl.ds(r0, S), :], osems[j % NX]
            ).wait()

        start_emb(0)
        start_x(0)
        start_x(1)

        for j in range(n_jobs):
            ci, b = divmod(j, B)
            xbuf, ebuf = xbufs[j % NX], ebufs[ci % 2]
            if j >= 2:
                wait_out(j - 2)
            if j + 2 < n_jobs:
                start_x(j + 2)
            nci = (j + 1) // B
            if nci != ci and nci < n_chunks:
                start_emb(nci)
            b_, r0 = rows(j)
            pltpu.make_async_copy(
                x_hbm.at[b_, pl.ds(T1 + r0, S), :], xbuf, xsems[j % NX]).wait()
            if b == 0:
                pltpu.make_async_copy(
                    emb_hbm.at[pl.ds(T1 + r0, S), :], ebuf,
                    esems[ci % 2]).wait()

            def radd(r, c, xbuf=xbuf, ebuf=ebuf):
                def cadd(i, c2):
                    for u in range(UNROLL):
                        sl = pl.ds((i * UNROLL + u) * 16, 16)
                        plsc.addupdate(xbuf.at[r, sl], ebuf[r, sl])
                    return c2

                return lax.fori_loop(0, D // (16 * UNROLL), cadd, c)

            lax.fori_loop(0, S, radd, 0)

            pltpu.async_copy(
                xbuf, out_hbm.at[b_, pl.ds(r0, S), :], osems[j % NX])

        wait_out(n_jobs - 2)
        wait_out(n_jobs - 1)

    return k


def kernel(x, emb):
    B, T, D = x.shape
    emb = emb[:T]
    sc_out = _make_sc_part(B, T, D)(x, emb)
    big = _tc_part(x, emb, B, T, D)
    return lax.dynamic_update_slice(big, sc_out, (0, T1, 0))


# speedup vs baseline: 7.8913x; 1.4401x over previous
import jax
import jax.numpy as jnp
from jax.experimental import pallas as pl
from jax.experimental.pallas import tpu as pltpu


def _add_kernel(x_ref, emb_ref, o_ref):
    o_ref[...] = x_ref[...] + emb_ref[...]


def kernel(x, emb):
    B, T, D = x.shape
    BT = 2048
    grid = (T // BT, B)
    out = pl.pallas_call(
        _add_kernel,
        grid=grid,
        in_specs=[
            pl.BlockSpec((1, BT, D), lambda t, b: (b, t, 0)),
            pl.BlockSpec((BT, D), lambda t, b: (t, 0)),
        ],
        out_specs=pl.BlockSpec((1, BT, D), lambda t, b: (b, t, 0)),
        out_shape=jax.ShapeDtypeStruct((B, T, D), x.dtype),
        compiler_params=pltpu.CompilerParams(
            dimension_semantics=("parallel", "parallel"),
            vmem_limit_bytes=128 * 1024 * 1024,
        ),
    )(x, emb[:T])
    return out
